# Initial kernel scaffold; baseline (speedup 1.0000x reference)
import jax
import jax.numpy as jnp
from jax.experimental import pallas as pl


def kernel(x, edge_index, edge_weight, W1, b1, W2, b2):
    raise NotImplementedError("write your pallas kernel here")


# speedup vs baseline: 6.8781x; 6.8781x over previous
import functools

import jax
import jax.numpy as jnp
from jax import lax
from jax.experimental import pallas as pl
from jax.experimental.pallas import tpu as pltpu
from jax.experimental.pallas import tpu_sc as plsc

NC = 2
NS = 16
NW = NC * NS
C = 128
LANES = 16


def _round_up(a, b):
    return (a + b - 1) // b * b


def _rsqrt16(x):
    i = lax.bitcast_convert_type(x, jnp.int32)
    i = jnp.int32(0x5F3759DF) - lax.shift_right_arithmetic(i, 1)
    y = lax.bitcast_convert_type(i, jnp.float32)
    for _ in range(3):
        y = y * (1.5 - 0.5 * x * y * y)
    return y


def _mesh():
    return plsc.VectorSubcoreMesh(
        core_axis_name="c", subcore_axis_name="s", num_cores=NC,
        num_subcores=NS)


def _make_deg_kernel(n_pad, epw):
    @functools.partial(
        pl.kernel,
        out_type=jax.ShapeDtypeStruct((NW, n_pad), jnp.float32),
        mesh=_mesh(),
        compiler_params=pltpu.CompilerParams(needs_layout_passes=False),
        scratch_types=[
            pltpu.VMEM((epw,), jnp.int32),
            pltpu.VMEM((epw,), jnp.float32),
            pltpu.VMEM((n_pad,), jnp.float32),
        ],
    )
    def deg_kernel(dst2, w2, degp, dstv, wv, acc):
        cid = lax.axis_index("c")
        sid = lax.axis_index("s")
        wid = sid * NC + cid
        pltpu.sync_copy(dst2.at[wid], dstv)
        pltpu.sync_copy(w2.at[wid], wv)

        def zero_body(i, _):
            acc[pl.ds(i * LANES, LANES)] = jnp.zeros((LANES,), jnp.float32)
            return 0
        lax.fori_loop(0, n_pad // LANES, zero_body, 0)

        def edge_body(j, _):
            d16 = dstv[pl.ds(j * LANES, LANES)]
            v16 = wv[pl.ds(j * LANES, LANES)]
            plsc.addupdate_scatter(acc, [d16], v16)
            return 0
        lax.fori_loop(0, epw // LANES, edge_body, 0)
        pltpu.sync_copy(acc, degp.at[wid])

    return deg_kernel


KB = 20


def _make_msg_kernel(n_pad, epw, h, first_layer):
    nch = epw // C
    nb = nch // KB
    rows_per_tile = n_pad // NS
    zchunks = rows_per_tile // C
    dchunk = n_pad // NW

    if first_layer:
        out_type = (
            jax.ShapeDtypeStruct((NC, n_pad, h), jnp.float32),
            jax.ShapeDtypeStruct((n_pad,), jnp.float32),
            jax.ShapeDtypeStruct((NW, epw), jnp.float32),
        )
    else:
        out_type = jax.ShapeDtypeStruct((NC, n_pad, h), jnp.float32)

    scratch = [
        pltpu.VMEM((KB, C), jnp.int32),
        pltpu.VMEM((KB, C), jnp.int32),
        pltpu.VMEM((KB * C,), jnp.float32),
        pltpu.VMEM((C, h), jnp.float32),
        pltpu.VMEM_SHARED((n_pad, h), jnp.float32),
        pltpu.SemaphoreType.DMA,
    ]
    if first_layer:
        scratch = scratch + [
            pltpu.VMEM((n_pad,), jnp.float32),
            pltpu.VMEM((rows_per_tile,), jnp.float32),
            pltpu.VMEM_SHARED((n_pad,), jnp.float32),
        ]

    def body(hmat, src3, dst3, *rest):
        if first_layer:
            degp, w2 = rest[0], rest[1]
            part_out, dis_out, norm_out = rest[2], rest[3], rest[4]
            srcv, dstv, normv, rows, acc_sh, sem, accdis, buf, dis_sh = \
                rest[5:]
        else:
            norm2 = rest[0]
            part_out = rest[1]
            srcv, dstv, normv, rows, acc_sh, sem = rest[2:]

        cid = lax.axis_index("c")
        sid = lax.axis_index("s")
        wid = sid * NC + cid
        row0 = sid * rows_per_tile

        def zrow(r, _):
            for q in range(h // LANES):
                rows[r, pl.ds(q * LANES, LANES)] = jnp.zeros(
                    (LANES,), jnp.float32)
            return 0
        lax.fori_loop(0, C, zrow, 0)
        for i in range(zchunks):
            pltpu.sync_copy(rows, acc_sh.at[pl.ds(row0 + i * C, C)])

        if first_layer:
            my = pl.ds(row0, rows_per_tile)

            def z2(i, _):
                accdis[pl.ds(row0 + i * LANES, LANES)] = jnp.zeros(
                    (LANES,), jnp.float32)
                return 0
            lax.fori_loop(0, rows_per_tile // LANES, z2, 0)

            def addp(p, _):
                pltpu.sync_copy(degp.at[p, my], buf)
                def inner(i, _):
                    sl = pl.ds(row0 + i * LANES, LANES)
                    accdis[sl] = accdis[sl] + buf[pl.ds(i * LANES, LANES)]
                    return 0
                lax.fori_loop(0, rows_per_tile // LANES, inner, 0)
                return 0
            lax.fori_loop(0, NW, addp, 0)

            def mkdis(i, _):
                sl = pl.ds(row0 + i * LANES, LANES)
                accdis[sl] = _rsqrt16(accdis[sl] + 1.0)
                return 0
            lax.fori_loop(0, rows_per_tile // LANES, mkdis, 0)

            pltpu.sync_copy(accdis.at[my], dis_sh.at[my])
            plsc.subcore_barrier()
            pltpu.sync_copy(dis_sh, accdis)
            dsl = pl.ds(cid * (n_pad // NC) + sid * dchunk, dchunk)
            pltpu.sync_copy(accdis.at[dsl], dis_out.at[dsl])
        else:
            plsc.subcore_barrier()


        def block(b, _):
            pltpu.sync_copy(src3.at[wid, b], srcv)
            pltpu.sync_copy(dst3.at[wid, b], dstv)
            esl = pl.ds(b * KB * C, KB * C)
            if first_layer:
                pltpu.sync_copy(w2.at[wid, esl], normv)

                def normj(jj, _):
                    for g in range(C // LANES):
                        base = jj * C + g * LANES
                        s16 = srcv[jj, pl.ds(g * LANES, LANES)]
                        d16 = dstv[jj, pl.ds(g * LANES, LANES)]
                        w16 = normv[pl.ds(base, LANES)]
                        n16 = (plsc.load_gather(accdis, [s16]) * w16 *
                               plsc.load_gather(accdis, [d16]))
                        normv[pl.ds(base, LANES)] = n16
                    return 0
                lax.fori_loop(0, KB, normj, 0)
                pltpu.sync_copy(normv, norm_out.at[wid, esl])
            else:
                pltpu.sync_copy(norm2.at[wid, esl], normv)

            def chunk(jj, _):
                pltpu.async_copy(hmat.at[srcv.at[jj]], rows, sem).wait()

                def scale(g, _):
                    nvec = normv[pl.ds(jj * C + g * LANES, LANES)]
                    for t in range(LANES):
                        nb = lax.gather(
                            nvec, jnp.full((LANES, 1), t, jnp.int32),
                            lax.GatherDimensionNumbers(
                                offset_dims=(), collapsed_slice_dims=(0,),
                                start_index_map=(0,)),
                            (1,),
                            mode=lax.GatherScatterMode.PROMISE_IN_BOUNDS)
                        e = g * LANES + t
                        for q in range(h // LANES):
                            sl = pl.ds(q * LANES, LANES)
                            rows[e, sl] = rows[e, sl] * nb
                    return 0
                lax.fori_loop(0, C // LANES, scale, 0)
                pltpu.sync_copy(rows, acc_sh.at[dstv.at[jj]], add=True)
                return 0
            lax.fori_loop(0, KB, chunk, 0)
            return 0
        lax.fori_loop(0, nb, block, 0)

        plsc.subcore_barrier()
        for i in range(zchunks):
            sl = pl.ds(row0 + i * C, C)
            pltpu.sync_copy(acc_sh.at[sl], part_out.at[cid, sl])

    return functools.partial(
        pl.kernel, out_type=out_type, mesh=_mesh(),
        compiler_params=pltpu.CompilerParams(needs_layout_passes=False),
        scratch_types=scratch)(body)


def _make_matmul(n_pad, d, h, bm):
    def body(xr, wr, outr):
        outr[...] = jnp.dot(xr[...], wr[...],
                            preferred_element_type=jnp.float32)
    return pl.pallas_call(
        body,
        grid=(n_pad // bm,),
        in_specs=[
            pl.BlockSpec((bm, d), lambda i: (i, 0)),
            pl.BlockSpec((d, h), lambda i: (0, 0)),
        ],
        out_specs=pl.BlockSpec((bm, h), lambda i: (i, 0)),
        out_shape=jax.ShapeDtypeStruct((n_pad, h), jnp.float32),
    )


def _make_combine(n_pad, h, bm, with_matmul):
    if with_matmul:
        def body(pr, dr, hr, br, wr, x1r, h2r):
            p = pr[0] + pr[1]
            dv = dr[...]
            x1 = jnp.maximum(p + dv * dv * hr[...] + br[...], 0.0)
            x1r[...] = x1
            h2r[...] = jnp.dot(x1, wr[...],
                               preferred_element_type=jnp.float32)
        in_specs = [
            pl.BlockSpec((NC, bm, h), lambda i: (0, i, 0)),
            pl.BlockSpec((bm, 1), lambda i: (i, 0)),
            pl.BlockSpec((bm, h), lambda i: (i, 0)),
            pl.BlockSpec((1, h), lambda i: (0, 0)),
            pl.BlockSpec((h, h), lambda i: (0, 0)),
        ]
        out_specs = [
            pl.BlockSpec((bm, h), lambda i: (i, 0)),
            pl.BlockSpec((bm, h), lambda i: (i, 0)),
        ]
        out_shape = [
            jax.ShapeDtypeStruct((n_pad, h), jnp.float32),
            jax.ShapeDtypeStruct((n_pad, h), jnp.float32),
        ]
    else:
        def body(pr, dr, hr, br, x1r):
            p = pr[0] + pr[1]
            dv = dr[...]
            x1r[...] = jnp.maximum(p + dv * dv * hr[...] + br[...], 0.0)
        in_specs = [
            pl.BlockSpec((NC, bm, h), lambda i: (0, i, 0)),
            pl.BlockSpec((bm, 1), lambda i: (i, 0)),
            pl.BlockSpec((bm, h), lambda i: (i, 0)),
            pl.BlockSpec((1, h), lambda i: (0, 0)),
        ]
        out_specs = pl.BlockSpec((bm, h), lambda i: (i, 0))
        out_shape = jax.ShapeDtypeStruct((n_pad, h), jnp.float32)
    return pl.pallas_call(
        body, grid=(n_pad // bm,), in_specs=in_specs,
        out_specs=out_specs, out_shape=out_shape)


def kernel(x, edge_index, edge_weight, W1, b1, W2, b2):
    n, d = x.shape
    h = W1.shape[1]
    e = edge_weight.shape[0]

    epw = _round_up(-(-e // NW), KB * C)
    e_pad = epw * NW
    n_pad = _round_up(n, NS * C)
    bm = 512

    src = edge_index[0]
    dst = edge_index[1]
    zpad_i = jnp.zeros((e_pad - e,), jnp.int32)
    nb = epw // (KB * C)
    src3 = jnp.concatenate([src, zpad_i]).reshape(NW, nb, KB, C)
    dst3 = jnp.concatenate([dst, zpad_i]).reshape(NW, nb, KB, C)
    dst2 = jnp.concatenate([dst, zpad_i]).reshape(NW, epw)
    w2 = jnp.concatenate(
        [edge_weight, jnp.zeros((e_pad - e,), jnp.float32)]).reshape(NW, epw)
    xp = jnp.concatenate(
        [x, jnp.zeros((n_pad - n, d), jnp.float32)], axis=0)

    deg_k = _make_deg_kernel(n_pad, epw)
    msg1_k = _make_msg_kernel(n_pad, epw, h, first_layer=True)
    msg2_k = _make_msg_kernel(n_pad, epw, h, first_layer=False)
    mm_k = _make_matmul(n_pad, d, h, bm)
    comb1_k = _make_combine(n_pad, h, bm, with_matmul=True)
    comb2_k = _make_combine(n_pad, h, bm, with_matmul=False)

    degp = deg_k(dst2, w2)
    h1 = mm_k(xp, W1)
    part1, dis, norm = msg1_k(h1, src3, dst3, degp, w2)
    dis2 = dis.reshape(n_pad, 1)
    x1, h2 = comb1_k(part1, dis2, h1, b1.reshape(1, h), W2)
    part2 = msg2_k(h2, src3, dst3, norm)
    x2 = comb2_k(part2, dis2, h2, b2.reshape(1, h))

    return jnp.concatenate([x1[:n], x2[:n]], axis=1)


# speedup vs baseline: 8.4294x; 1.2255x over previous
import functools

import jax
import jax.numpy as jnp
from jax import lax
from jax.experimental import pallas as pl
from jax.experimental.pallas import tpu as pltpu
from jax.experimental.pallas import tpu_sc as plsc

NC = 2
NS = 16
NW = NC * NS
C = 128
KB = 20
LANES = 16


def _round_up(a, b):
    return (a + b - 1) // b * b


def _rsqrt16(x):
    i = lax.bitcast_convert_type(x, jnp.int32)
    i = jnp.int32(0x5F3759DF) - lax.shift_right_arithmetic(i, 1)
    y = lax.bitcast_convert_type(i, jnp.float32)
    for _ in range(3):
        y = y * (1.5 - 0.5 * x * y * y)
    return y


def _mesh():
    return plsc.VectorSubcoreMesh(
        core_axis_name="c", subcore_axis_name="s", num_cores=NC,
        num_subcores=NS)


_SC_PARAMS = dict(
    compiler_params=pltpu.CompilerParams(needs_layout_passes=False))


def _make_deg_kernel(n_pad, epw):
    @functools.partial(
        pl.kernel,
        out_type=jax.ShapeDtypeStruct((NW, n_pad), jnp.float32),
        mesh=_mesh(),
        compiler_params=pltpu.CompilerParams(needs_layout_passes=False),
        scratch_types=[
            pltpu.VMEM((epw,), jnp.int32),
            pltpu.VMEM((epw,), jnp.float32),
            pltpu.VMEM((n_pad,), jnp.float32),
        ],
    )
    def deg_kernel(dst2, w2, degp, dstv, wv, acc):
        cid = lax.axis_index("c")
        sid = lax.axis_index("s")
        wid = sid * NC + cid
        pltpu.sync_copy(dst2.at[wid], dstv)
        pltpu.sync_copy(w2.at[wid], wv)

        def zero_body(i, _):
            acc[pl.ds(i * LANES, LANES)] = jnp.zeros((LANES,), jnp.float32)
            return 0
        lax.fori_loop(0, n_pad // LANES, zero_body, 0)

        def edge_body(j, _):
            d16 = dstv[pl.ds(j * LANES, LANES)]
            v16 = wv[pl.ds(j * LANES, LANES)]
            plsc.addupdate_scatter(acc, [d16], v16)
            return 0
        lax.fori_loop(0, epw // LANES, edge_body, 0)
        pltpu.sync_copy(acc, degp.at[wid])

    return deg_kernel


def _make_norm_kernel(n_pad, epw32):
    @functools.partial(
        pl.kernel,
        out_type=(
            jax.ShapeDtypeStruct((n_pad,), jnp.float32),
            jax.ShapeDtypeStruct((NW, epw32), jnp.float32),
        ),
        mesh=_mesh(),
        scratch_types=[
            pltpu.VMEM((epw32,), jnp.int32),
            pltpu.VMEM((epw32,), jnp.int32),
            pltpu.VMEM((epw32,), jnp.float32),
            pltpu.VMEM((n_pad,), jnp.float32),
            pltpu.VMEM((n_pad // NS,), jnp.float32),
            pltpu.VMEM_SHARED((n_pad,), jnp.float32),
        ],
        **_SC_PARAMS,
    )
    def norm_kernel(degp, src2, dst2, w2, dis_out, norm_out,
                    srcv, dstv, wv, accdis, buf, dis_sh):
        cid = lax.axis_index("c")
        sid = lax.axis_index("s")
        wid = sid * NC + cid
        rows_per_tile = n_pad // NS
        row0 = sid * rows_per_tile
        my = pl.ds(row0, rows_per_tile)

        pltpu.sync_copy(src2.at[wid], srcv)
        pltpu.sync_copy(dst2.at[wid], dstv)
        pltpu.sync_copy(w2.at[wid], wv)

        def z2(i, _):
            accdis[pl.ds(row0 + i * LANES, LANES)] = jnp.zeros(
                (LANES,), jnp.float32)
            return 0
        lax.fori_loop(0, rows_per_tile // LANES, z2, 0)

        def addp(p, _):
            pltpu.sync_copy(degp.at[p, my], buf)
            def inner(i, _):
                sl = pl.ds(row0 + i * LANES, LANES)
                accdis[sl] = accdis[sl] + buf[pl.ds(i * LANES, LANES)]
                return 0
            lax.fori_loop(0, rows_per_tile // LANES, inner, 0)
            return 0
        lax.fori_loop(0, NW, addp, 0)

        def mkdis(i, _):
            sl = pl.ds(row0 + i * LANES, LANES)
            accdis[sl] = _rsqrt16(accdis[sl] + 1.0)
            return 0
        lax.fori_loop(0, rows_per_tile // LANES, mkdis, 0)

        pltpu.sync_copy(accdis.at[my], dis_sh.at[my])
        plsc.subcore_barrier()
        pltpu.sync_copy(dis_sh, accdis)
        dchunk = n_pad // NW
        dsl = pl.ds(cid * (n_pad // NC) + sid * dchunk, dchunk)
        pltpu.sync_copy(accdis.at[dsl], dis_out.at[dsl])

        def normk(k, _):
            sl = pl.ds(k * LANES, LANES)
            s16 = srcv[sl]
            d16 = dstv[sl]
            wv[sl] = (plsc.load_gather(accdis, [s16]) * wv[sl] *
                      plsc.load_gather(accdis, [d16]))
            return 0
        lax.fori_loop(0, epw32 // LANES, normk, 0)
        pltpu.sync_copy(wv, norm_out.at[wid])

    return norm_kernel


def _make_msg_kernel(n_pad, epw, h):
    nch = epw // C
    nb = nch // KB
    rows_per_tile = n_pad // NS
    zchunks = rows_per_tile // C

    scratch = [
        pltpu.VMEM((KB, C), jnp.int32),
        pltpu.VMEM((KB, C), jnp.int32),
        pltpu.VMEM((KB * C,), jnp.float32),
        [pltpu.VMEM((C, h), jnp.float32) for _ in range(2)],
        pltpu.VMEM_SHARED((n_pad, h), jnp.float32),
        [pltpu.SemaphoreType.DMA for _ in range(2)],
        [pltpu.SemaphoreType.DMA for _ in range(2)],
    ]

    @functools.partial(
        pl.kernel,
        out_type=jax.ShapeDtypeStruct((NC, n_pad, h), jnp.float32),
        mesh=_mesh(),
        scratch_types=scratch,
        **_SC_PARAMS,
    )
    def msg_kernel(hmat, src3, dst3, norm2, part_out,
                   srcv, dstv, normv, rows, acc_sh, gsem, ssem):
        cid = lax.axis_index("c")
        sid = lax.axis_index("s")
        wid = sid * NC + cid
        row0 = sid * rows_per_tile

        def zrow(r, _):
            for q in range(h // LANES):
                rows[0][r, pl.ds(q * LANES, LANES)] = jnp.zeros(
                    (LANES,), jnp.float32)
            return 0
        lax.fori_loop(0, C, zrow, 0)
        for i in range(zchunks):
            pltpu.sync_copy(rows[0], acc_sh.at[pl.ds(row0 + i * C, C)])
        plsc.subcore_barrier()

        def gissue(jj, b):
            pltpu.async_copy(hmat.at[srcv.at[jj]], rows[b], gsem[b])

        def gwait(jj, b):
            pltpu.make_async_copy(
                hmat.at[srcv.at[jj]], rows[b], gsem[b]).wait()

        def sissue(jj, b):
            pltpu.async_copy(
                rows[b], acc_sh.at[dstv.at[jj]], ssem[b], add=True)

        def swait(jj, b):
            pltpu.make_async_copy(
                rows[b], acc_sh.at[dstv.at[jj]], ssem[b]).wait()

        def scale(jj, b):
            def sg(g, _):
                nvec = normv[pl.ds(jj * C + g * LANES, LANES)]
                for t in range(LANES):
                    nb_ = lax.gather(
                        nvec, jnp.full((LANES, 1), t, jnp.int32),
                        lax.GatherDimensionNumbers(
                            offset_dims=(), collapsed_slice_dims=(0,),
                            start_index_map=(0,)),
                        (1,),
                        mode=lax.GatherScatterMode.PROMISE_IN_BOUNDS)
                    e = g * LANES + t
                    for q in range(h // LANES):
                        sl = pl.ds(q * LANES, LANES)
                        rows[b][e, sl] = rows[b][e, sl] * nb_
                return 0
            lax.fori_loop(0, C // LANES, sg, 0)

        def step(jj, b, first, last):
            gwait(jj, b)
            if not first:
                swait(jj - 1, 1 - b)
            if not last:
                gissue(jj + 1, 1 - b)
            scale(jj, b)
            sissue(jj, b)

        def block(bi, _):
            pltpu.sync_copy(src3.at[wid, bi], srcv)
            pltpu.sync_copy(dst3.at[wid, bi], dstv)
            pltpu.sync_copy(norm2.at[wid, pl.ds(bi * KB * C, KB * C)],
                            normv)
            gissue(0, 0)
            step(0, 0, True, False)
            step(1, 1, False, False)

            def steady(j2, _):
                jj = j2 * 2
                step(jj, 0, False, False)
                step(jj + 1, 1, False, False)
                return 0
            lax.fori_loop(1, KB // 2 - 1, steady, 0)
            step(KB - 2, 0, False, False)
            step(KB - 1, 1, False, True)
            swait(KB - 1, 1)
            return 0
        lax.fori_loop(0, nb, block, 0)

        plsc.subcore_barrier()
        for i in range(zchunks):
            sl = pl.ds(row0 + i * C, C)
            pltpu.sync_copy(acc_sh.at[sl], part_out.at[cid, sl])

    return msg_kernel


def _make_matmul(n_pad, d, h, bm):
    def body(xr, wr, outr):
        outr[...] = jnp.dot(xr[...], wr[...],
                            preferred_element_type=jnp.float32)
    return pl.pallas_call(
        body,
        grid=(n_pad // bm,),
        in_specs=[
            pl.BlockSpec((bm, d), lambda i: (i, 0)),
            pl.BlockSpec((d, h), lambda i: (0, 0)),
        ],
        out_specs=pl.BlockSpec((bm, h), lambda i: (i, 0)),
        out_shape=jax.ShapeDtypeStruct((n_pad, h), jnp.float32),
    )


def _make_combine(n_pad, h, bm, with_matmul):
    if with_matmul:
        def body(pr, dr, hr, br, wr, x1r, h2r):
            p = pr[0] + pr[1]
            dv = dr[...]
            x1 = jnp.maximum(p + dv * dv * hr[...] + br[...], 0.0)
            x1r[...] = x1
            h2r[...] = jnp.dot(x1, wr[...],
                               preferred_element_type=jnp.float32)
        in_specs = [
            pl.BlockSpec((NC, bm, h), lambda i: (0, i, 0)),
            pl.BlockSpec((bm, 1), lambda i: (i, 0)),
            pl.BlockSpec((bm, h), lambda i: (i, 0)),
            pl.BlockSpec((1, h), lambda i: (0, 0)),
            pl.BlockSpec((h, h), lambda i: (0, 0)),
        ]
        out_specs = [
            pl.BlockSpec((bm, h), lambda i: (i, 0)),
            pl.BlockSpec((bm, h), lambda i: (i, 0)),
        ]
        out_shape = [
            jax.ShapeDtypeStruct((n_pad, h), jnp.float32),
            jax.ShapeDtypeStruct((n_pad, h), jnp.float32),
        ]
    else:
        def body(pr, dr, hr, br, x1r):
            p = pr[0] + pr[1]
            dv = dr[...]
            x1r[...] = jnp.maximum(p + dv * dv * hr[...] + br[...], 0.0)
        in_specs = [
            pl.BlockSpec((NC, bm, h), lambda i: (0, i, 0)),
            pl.BlockSpec((bm, 1), lambda i: (i, 0)),
            pl.BlockSpec((bm, h), lambda i: (i, 0)),
            pl.BlockSpec((1, h), lambda i: (0, 0)),
        ]
        out_specs = pl.BlockSpec((bm, h), lambda i: (i, 0))
        out_shape = jax.ShapeDtypeStruct((n_pad, h), jnp.float32)
    return pl.pallas_call(
        body, grid=(n_pad // bm,), in_specs=in_specs,
        out_specs=out_specs, out_shape=out_shape)


def kernel(x, edge_index, edge_weight, W1, b1, W2, b2):
    n, d = x.shape
    h = W1.shape[1]
    e = edge_weight.shape[0]

    epw = _round_up(-(-e // NW), KB * C)
    e_pad = epw * NW
    n_pad = _round_up(n, NS * C)
    nb = epw // (KB * C)
    bm = 512

    src = edge_index[0]
    dst = edge_index[1]
    zpad_i = jnp.zeros((e_pad - e,), jnp.int32)
    srcf = jnp.concatenate([src, zpad_i])
    dstf = jnp.concatenate([dst, zpad_i])
    wf = jnp.concatenate([edge_weight, jnp.zeros((e_pad - e,), jnp.float32)])
    src3 = srcf.reshape(NW, nb, KB, C)
    dst3 = dstf.reshape(NW, nb, KB, C)
    src2 = srcf.reshape(NW, epw)
    dst2 = dstf.reshape(NW, epw)
    w2 = wf.reshape(NW, epw)
    xp = jnp.concatenate([x, jnp.zeros((n_pad - n, d), jnp.float32)], axis=0)

    deg_k = _make_deg_kernel(n_pad, epw)
    norm_k = _make_norm_kernel(n_pad, epw)
    msg_k = _make_msg_kernel(n_pad, epw, h)
    mm_k = _make_matmul(n_pad, d, h, bm)
    comb1_k = _make_combine(n_pad, h, bm, with_matmul=True)
    comb2_k = _make_combine(n_pad, h, bm, with_matmul=False)

    degp = deg_k(dst2, w2)
    h1 = mm_k(xp, W1)
    dis, norm = norm_k(degp, src2, dst2, w2)
    part1 = msg_k(h1, src3, dst3, norm)
    dis2 = dis.reshape(n_pad, 1)
    x1, h2 = comb1_k(part1, dis2, h1, b1.reshape(1, h), W2)
    part2 = msg_k(h2, src3, dst3, norm)
    x2 = comb2_k(part2, dis2, h2, b2.reshape(1, h))

    return jnp.concatenate([x1[:n], x2[:n]], axis=1)


# speedup vs baseline: 8.6222x; 1.0229x over previous
import functools

import jax
import jax.numpy as jnp
from jax import lax
from jax.experimental import pallas as pl
from jax.experimental.pallas import tpu as pltpu
from jax.experimental.pallas import tpu_sc as plsc

NC = 2
NS = 16
NW = NC * NS
C = 128
KB = 20
LANES = 16


def _round_up(a, b):
    return (a + b - 1) // b * b


def _rsqrt16(x):
    i = lax.bitcast_convert_type(x, jnp.int32)
    i = jnp.int32(0x5F3759DF) - lax.shift_right_arithmetic(i, 1)
    y = lax.bitcast_convert_type(i, jnp.float32)
    for _ in range(3):
        y = y * (1.5 - 0.5 * x * y * y)
    return y


def _mesh():
    return plsc.VectorSubcoreMesh(
        core_axis_name="c", subcore_axis_name="s", num_cores=NC,
        num_subcores=NS)


_SC_PARAMS = dict(
    compiler_params=pltpu.CompilerParams(needs_layout_passes=False))


def _make_deg_kernel(n_pad, epw):
    @functools.partial(
        pl.kernel,
        out_type=jax.ShapeDtypeStruct((NW, n_pad), jnp.float32),
        mesh=_mesh(),
        compiler_params=pltpu.CompilerParams(needs_layout_passes=False),
        scratch_types=[
            pltpu.VMEM((epw,), jnp.int32),
            pltpu.VMEM((epw,), jnp.float32),
            pltpu.VMEM((n_pad,), jnp.float32),
        ],
    )
    def deg_kernel(dst2, w2, degp, dstv, wv, acc):
        cid = lax.axis_index("c")
        sid = lax.axis_index("s")
        wid = sid * NC + cid
        pltpu.sync_copy(dst2.at[wid], dstv)
        pltpu.sync_copy(w2.at[wid], wv)

        def zero_body(i, _):
            acc[pl.ds(i * LANES, LANES)] = jnp.zeros((LANES,), jnp.float32)
            return 0
        lax.fori_loop(0, n_pad // LANES, zero_body, 0)

        def edge_body(j, _):
            d16 = dstv[pl.ds(j * LANES, LANES)]
            v16 = wv[pl.ds(j * LANES, LANES)]
            plsc.addupdate_scatter(acc, [d16], v16)
            return 0
        lax.fori_loop(0, epw // LANES, edge_body, 0)
        pltpu.sync_copy(acc, degp.at[wid])

    return deg_kernel


def _make_norm_kernel(n_pad, epw32):
    @functools.partial(
        pl.kernel,
        out_type=(
            jax.ShapeDtypeStruct((n_pad,), jnp.float32),
            jax.ShapeDtypeStruct((NW, epw32), jnp.float32),
        ),
        mesh=_mesh(),
        scratch_types=[
            pltpu.VMEM((epw32,), jnp.int32),
            pltpu.VMEM((epw32,), jnp.int32),
            pltpu.VMEM((epw32,), jnp.float32),
            pltpu.VMEM((n_pad,), jnp.float32),
            pltpu.VMEM((n_pad // NS,), jnp.float32),
            pltpu.VMEM_SHARED((n_pad,), jnp.float32),
        ],
        **_SC_PARAMS,
    )
    def norm_kernel(degp, src2, dst2, w2, dis_out, norm_out,
                    srcv, dstv, wv, accdis, buf, dis_sh):
        cid = lax.axis_index("c")
        sid = lax.axis_index("s")
        wid = sid * NC + cid
        rows_per_tile = n_pad // NS
        row0 = sid * rows_per_tile
        my = pl.ds(row0, rows_per_tile)

        pltpu.sync_copy(src2.at[wid], srcv)
        pltpu.sync_copy(dst2.at[wid], dstv)
        pltpu.sync_copy(w2.at[wid], wv)

        def z2(i, _):
            accdis[pl.ds(row0 + i * LANES, LANES)] = jnp.zeros(
                (LANES,), jnp.float32)
            return 0
        lax.fori_loop(0, rows_per_tile // LANES, z2, 0)

        def addp(p, _):
            pltpu.sync_copy(degp.at[p, my], buf)
            def inner(i, _):
                sl = pl.ds(row0 + i * LANES, LANES)
                accdis[sl] = accdis[sl] + buf[pl.ds(i * LANES, LANES)]
                return 0
            lax.fori_loop(0, rows_per_tile // LANES, inner, 0)
            return 0
        lax.fori_loop(0, NW, addp, 0)

        def mkdis(i, _):
            sl = pl.ds(row0 + i * LANES, LANES)
            accdis[sl] = _rsqrt16(accdis[sl] + 1.0)
            return 0
        lax.fori_loop(0, rows_per_tile // LANES, mkdis, 0)

        pltpu.sync_copy(accdis.at[my], dis_sh.at[my])
        plsc.subcore_barrier()
        pltpu.sync_copy(dis_sh, accdis)
        dchunk = n_pad // NW
        dsl = pl.ds(cid * (n_pad // NC) + sid * dchunk, dchunk)
        pltpu.sync_copy(accdis.at[dsl], dis_out.at[dsl])

        def normk(k, _):
            sl = pl.ds(k * LANES, LANES)
            s16 = srcv[sl]
            d16 = dstv[sl]
            wv[sl] = (plsc.load_gather(accdis, [s16]) * wv[sl] *
                      plsc.load_gather(accdis, [d16]))
            return 0
        lax.fori_loop(0, epw32 // LANES, normk, 0)
        pltpu.sync_copy(wv, norm_out.at[wid])

    return norm_kernel


def _make_msg_kernel(n_pad, epw, h):
    nch = epw // C
    nb = nch // KB
    rows_per_tile = n_pad // NS
    zchunks = rows_per_tile // C

    scratch = [
        pltpu.VMEM((KB, C), jnp.int32),
        pltpu.VMEM((KB, C), jnp.int32),
        pltpu.VMEM((KB * C,), jnp.float32),
        [pltpu.VMEM((C, h), jnp.float32) for _ in range(2)],
        pltpu.VMEM_SHARED((n_pad, h), jnp.float32),
        [pltpu.SemaphoreType.DMA for _ in range(2)],
        [pltpu.SemaphoreType.DMA for _ in range(2)],
    ]

    @functools.partial(
        pl.kernel,
        out_type=jax.ShapeDtypeStruct((NC, n_pad, h), jnp.float32),
        mesh=_mesh(),
        scratch_types=scratch,
        **_SC_PARAMS,
    )
    def msg_kernel(hmat, src3, dst3, norm2, part_out,
                   srcv, dstv, normv, rows, acc_sh, gsem, ssem):
        cid = lax.axis_index("c")
        sid = lax.axis_index("s")
        wid = sid * NC + cid
        row0 = sid * rows_per_tile

        def zrow(r, _):
            for q in range(h // LANES):
                rows[0][r, pl.ds(q * LANES, LANES)] = jnp.zeros(
                    (LANES,), jnp.float32)
            return 0
        lax.fori_loop(0, C, zrow, 0)
        for i in range(zchunks):
            pltpu.sync_copy(rows[0], acc_sh.at[pl.ds(row0 + i * C, C)])
        plsc.subcore_barrier()

        def gissue(jj, b):
            pltpu.async_copy(hmat.at[srcv.at[jj]], rows[b], gsem[b])

        def gwait(jj, b):
            pltpu.make_async_copy(
                hmat.at[srcv.at[jj]], rows[b], gsem[b]).wait()

        def sissue(jj, b):
            pltpu.async_copy(
                rows[b], acc_sh.at[dstv.at[jj]], ssem[b], add=True)

        def swait(jj, b):
            pltpu.make_async_copy(
                rows[b], acc_sh.at[dstv.at[jj]], ssem[b]).wait()

        def scale(jj, b):
            def sg(g, _):
                nvec = normv[pl.ds(jj * C + g * LANES, LANES)]
                for t in range(LANES):
                    nb_ = lax.gather(
                        nvec, jnp.full((LANES, 1), t, jnp.int32),
                        lax.GatherDimensionNumbers(
                            offset_dims=(), collapsed_slice_dims=(0,),
                            start_index_map=(0,)),
                        (1,),
                        mode=lax.GatherScatterMode.PROMISE_IN_BOUNDS)
                    e = g * LANES + t
                    for q in range(h // LANES):
                        sl = pl.ds(q * LANES, LANES)
                        rows[b][e, sl] = rows[b][e, sl] * nb_
                return 0
            lax.fori_loop(0, C // LANES, sg, 0)

        def step(jj, b, first, last):
            if not first:
                swait(jj - 1, 1 - b)
            if not last:
                gissue(jj + 1, 1 - b)
            gwait(jj, b)
            scale(jj, b)
            sissue(jj, b)

        def block(bi, _):
            pltpu.sync_copy(src3.at[wid, bi], srcv)
            pltpu.sync_copy(dst3.at[wid, bi], dstv)
            pltpu.sync_copy(norm2.at[wid, pl.ds(bi * KB * C, KB * C)],
                            normv)
            gissue(0, 0)
            step(0, 0, True, False)
            step(1, 1, False, False)

            def steady(j2, _):
                jj = j2 * 2
                step(jj, 0, False, False)
                step(jj + 1, 1, False, False)
                return 0
            lax.fori_loop(1, KB // 2 - 1, steady, 0)
            step(KB - 2, 0, False, False)
            step(KB - 1, 1, False, True)
            swait(KB - 1, 1)
            return 0
        lax.fori_loop(0, nb, block, 0)

        plsc.subcore_barrier()
        for i in range(zchunks):
            sl = pl.ds(row0 + i * C, C)
            pltpu.sync_copy(acc_sh.at[sl], part_out.at[cid, sl])

    return msg_kernel


def _make_matmul(n_pad, d, h, bm):
    def body(xr, wr, outr):
        outr[...] = jnp.dot(xr[...], wr[...],
                            preferred_element_type=jnp.float32)
    return pl.pallas_call(
        body,
        grid=(n_pad // bm,),
        in_specs=[
            pl.BlockSpec((bm, d), lambda i: (i, 0)),
            pl.BlockSpec((d, h), lambda i: (0, 0)),
        ],
        out_specs=pl.BlockSpec((bm, h), lambda i: (i, 0)),
        out_shape=jax.ShapeDtypeStruct((n_pad, h), jnp.float32),
    )


def _make_combine(n_pad, h, bm, with_matmul):
    if with_matmul:
        def body(pr, dr, hr, br, wr, x1r, h2r):
            p = pr[0] + pr[1]
            dv = dr[...]
            x1 = jnp.maximum(p + dv * dv * hr[...] + br[...], 0.0)
            x1r[...] = x1
            h2r[...] = jnp.dot(x1, wr[...],
                               preferred_element_type=jnp.float32)
        in_specs = [
            pl.BlockSpec((NC, bm, h), lambda i: (0, i, 0)),
            pl.BlockSpec((bm, 1), lambda i: (i, 0)),
            pl.BlockSpec((bm, h), lambda i: (i, 0)),
            pl.BlockSpec((1, h), lambda i: (0, 0)),
            pl.BlockSpec((h, h), lambda i: (0, 0)),
        ]
        out_specs = [
            pl.BlockSpec((bm, h), lambda i: (i, 0)),
            pl.BlockSpec((bm, h), lambda i: (i, 0)),
        ]
        out_shape = [
            jax.ShapeDtypeStruct((n_pad, h), jnp.float32),
            jax.ShapeDtypeStruct((n_pad, h), jnp.float32),
        ]
    else:
        def body(pr, dr, hr, br, x1r):
            p = pr[0] + pr[1]
            dv = dr[...]
            x1r[...] = jnp.maximum(p + dv * dv * hr[...] + br[...], 0.0)
        in_specs = [
            pl.BlockSpec((NC, bm, h), lambda i: (0, i, 0)),
            pl.BlockSpec((bm, 1), lambda i: (i, 0)),
            pl.BlockSpec((bm, h), lambda i: (i, 0)),
            pl.BlockSpec((1, h), lambda i: (0, 0)),
        ]
        out_specs = pl.BlockSpec((bm, h), lambda i: (i, 0))
        out_shape = jax.ShapeDtypeStruct((n_pad, h), jnp.float32)
    return pl.pallas_call(
        body, grid=(n_pad // bm,), in_specs=in_specs,
        out_specs=out_specs, out_shape=out_shape)


def kernel(x, edge_index, edge_weight, W1, b1, W2, b2):
    n, d = x.shape
    h = W1.shape[1]
    e = edge_weight.shape[0]

    epw = _round_up(-(-e // NW), KB * C)
    e_pad = epw * NW
    n_pad = _round_up(n, NS * C)
    nb = epw // (KB * C)
    bm = 512

    src = edge_index[0]
    dst = edge_index[1]
    zpad_i = jnp.zeros((e_pad - e,), jnp.int32)
    srcf = jnp.concatenate([src, zpad_i])
    dstf = jnp.concatenate([dst, zpad_i])
    wf = jnp.concatenate([edge_weight, jnp.zeros((e_pad - e,), jnp.float32)])
    src3 = srcf.reshape(NW, nb, KB, C)
    dst3 = dstf.reshape(NW, nb, KB, C)
    src2 = srcf.reshape(NW, epw)
    dst2 = dstf.reshape(NW, epw)
    w2 = wf.reshape(NW, epw)
    xp = jnp.concatenate([x, jnp.zeros((n_pad - n, d), jnp.float32)], axis=0)

    deg_k = _make_deg_kernel(n_pad, epw)
    norm_k = _make_norm_kernel(n_pad, epw)
    msg_k = _make_msg_kernel(n_pad, epw, h)
    mm_k = _make_matmul(n_pad, d, h, bm)
    comb1_k = _make_combine(n_pad, h, bm, with_matmul=True)
    comb2_k = _make_combine(n_pad, h, bm, with_matmul=False)

    degp = deg_k(dst2, w2)
    h1 = mm_k(xp, W1)
    dis, norm = norm_k(degp, src2, dst2, w2)
    part1 = msg_k(h1, src3, dst3, norm)
    dis2 = dis.reshape(n_pad, 1)
    x1, h2 = comb1_k(part1, dis2, h1, b1.reshape(1, h), W2)
    part2 = msg_k(h2, src3, dst3, norm)
    x2 = comb2_k(part2, dis2, h2, b2.reshape(1, h))

    return jnp.concatenate([x1[:n], x2[:n]], axis=1)
